# baseline (speedup 1.0000x reference)
import functools

import jax
import jax.numpy as jnp
from jax import lax
from jax.experimental import pallas as pl
from jax.experimental.pallas import tpu as pltpu
from jax.experimental.pallas import tpu_sc as plsc

_D = 64
_W = 2 * _D
_SCALE = 64 ** 0.5
_LANES = 16

_info = plsc.get_sparse_core_info()
_NC, _NS = _info.num_cores, _info.num_subcores
_NW = _NC * _NS


def _make_stage(n_rows: int, blk: int = 512):
    grid = -(-n_rows // blk)

    @functools.partial(
        pl.pallas_call,
        grid=(grid,),
        in_specs=[pl.BlockSpec((_D, blk), lambda g: (0, g))],
        out_specs=pl.BlockSpec((blk, _W), lambda g: (g, 0)),
        out_shape=jax.ShapeDtypeStruct((n_rows, _W), jnp.float32),
    )
    def stage(tt_ref, out_ref):
        y = tt_ref[...].T * _SCALE
        out_ref[...] = jnp.concatenate(
            [y, jnp.zeros((blk, _D), jnp.float32)], axis=1)

    return stage


def _make_gather(n_batch: int, n_time: int, chunk_rows: int):
    n_idx = n_batch * n_time
    chunk = chunk_rows * n_time
    assert n_idx % (_NW * chunk) == 0
    per_w = n_idx // _NW
    rows_w = per_w // n_time
    n_chunks = per_w // chunk
    assert n_chunks % 2 == 0 and n_chunks >= 4
    mesh = plsc.VectorSubcoreMesh(core_axis_name="c", subcore_axis_name="s")

    @functools.partial(
        pl.kernel,
        mesh=mesh,
        out_type=jax.ShapeDtypeStruct((n_batch, n_time, _W), jnp.float32),
        scratch_types=[
            pltpu.VMEM((chunk,), jnp.int32),
            pltpu.VMEM((chunk,), jnp.int32),
            pltpu.VMEM((chunk,), jnp.int32),
            pltpu.VMEM((chunk,), jnp.int32),
            pltpu.VMEM((chunk, _D), jnp.float32),
            pltpu.VMEM((chunk, _D), jnp.float32),
            pltpu.SemaphoreType.DMA,
            pltpu.SemaphoreType.DMA,
            pltpu.SemaphoreType.DMA,
            pltpu.SemaphoreType.DMA,
            pltpu.SemaphoreType.DMA,
            pltpu.SemaphoreType.DMA,
        ],
        compiler_params=pltpu.CompilerParams(use_tc_tiling_on_sc=False),
    )
    def gather_kernel(idx_hbm, table_hbm, out_hbm,
                      i0, i1, q0, q1, r0, r1, is0, is1, gs0, gs1, ss0, ss1):
        ibuf, qbuf, rbuf = (i0, i1), (q0, q1), (r0, r1)
        isem, gsem, ssem = (is0, is1), (gs0, gs1), (ss0, ss1)
        wid = lax.axis_index("s") * _NC + lax.axis_index("c")
        base = wid * per_w

        def off(g):
            return base + g * chunk

        def idx_start(g, b):
            pltpu.async_copy(idx_hbm.at[pl.ds(off(g), chunk)], ibuf[b], isem[b])

        def idx_wait(b):
            pltpu.make_async_copy(
                idx_hbm.at[pl.ds(base, chunk)], ibuf[b], isem[b]).wait()

        def gather_start(b):
            def dbl(j, carry):
                sl = pl.ds(j * _LANES, _LANES)
                qbuf[b][sl] = ibuf[b][sl] * 2
                return carry

            lax.fori_loop(0, chunk // _LANES, dbl, 0)
            pltpu.async_copy(table_hbm.at[qbuf[b]], rbuf[b], gsem[b])

        def gather_wait(b):
            pltpu.make_async_copy(
                table_hbm.at[qbuf[b]], rbuf[b], gsem[b]).wait()

        def scatter_start(g, b):
            row0 = wid * rows_w + g * chunk_rows
            for k in range(chunk_rows):
                pltpu.async_copy(
                    rbuf[b].at[pl.ds(k * n_time, n_time)],
                    out_hbm.at[row0 + k, :, pl.ds(0, _D)], ssem[b])

        def scatter_wait(b):
            for _ in range(chunk_rows):
                pltpu.make_async_copy(
                    rbuf[b].at[pl.ds(0, n_time)],
                    out_hbm.at[0, :, pl.ds(0, _D)], ssem[b]).wait()

        idx_start(0, 0)
        idx_start(1, 1)
        idx_wait(0)
        gather_start(0)

        def body(i, carry):
            for b in range(2):
                g = 2 * i + b
                nb = 1 - b

                @pl.when(g + 1 < n_chunks)
                def _():
                    idx_wait(nb)

                    @pl.when(g >= 1)
                    def _():
                        scatter_wait(nb)

                    gather_start(nb)

                gather_wait(b)

                @pl.when(g + 2 < n_chunks)
                def _():
                    idx_start(g + 2, b)

                scatter_start(g, b)
            return carry

        lax.fori_loop(0, n_chunks // 2, body, 0)
        scatter_wait(0)
        scatter_wait(1)

    return gather_kernel


def kernel(inp_tokens, emb_table):
    n_batch, n_time = inp_tokens.shape
    idx = inp_tokens.reshape(-1).astype(jnp.int32)
    table_dup = _make_stage(emb_table.shape[0])(emb_table.T)
    table_dup = table_dup.reshape(2 * emb_table.shape[0], _D)
    out128 = _make_gather(n_batch, n_time, 2)(idx, table_dup)
    return out128[:, :, :_D]


# speedup vs baseline: 1.6123x; 1.6123x over previous
import functools

import jax
import jax.numpy as jnp
from jax import lax
from jax.experimental import pallas as pl
from jax.experimental.pallas import tpu as pltpu
from jax.experimental.pallas import tpu_sc as plsc

_D = 64
_W = 2 * _D
_SCALE = 64 ** 0.5
_LANES = 16

_info = plsc.get_sparse_core_info()
_NC, _NS = _info.num_cores, _info.num_subcores
_NW = _NC * _NS


def _make_stage(n_rows: int, blk: int = 2048):
    grid = -(-n_rows // blk)

    @functools.partial(
        pl.pallas_call,
        grid=(grid,),
        in_specs=[pl.BlockSpec((_D, blk), lambda g: (0, g))],
        out_specs=pl.BlockSpec((blk, _W), lambda g: (g, 0)),
        out_shape=jax.ShapeDtypeStruct((n_rows, _W), jnp.float32),
    )
    def stage(tt_ref, out_ref):
        y = lax.dot_general(
            tt_ref[...], jnp.eye(_D, dtype=jnp.float32),
            (((0,), (0,)), ((), ())),
            precision=lax.Precision.HIGHEST,
            preferred_element_type=jnp.float32)
        out_ref[...] = jnp.concatenate(
            [y * _SCALE, jnp.zeros((blk, _D), jnp.float32)], axis=1)

    return stage


def _make_gather(n_batch: int, n_time: int, chunk_rows: int):
    n_idx = n_batch * n_time
    chunk = chunk_rows * n_time
    assert n_idx % (_NW * chunk) == 0
    per_w = n_idx // _NW
    rows_w = per_w // n_time
    n_chunks = per_w // chunk
    assert n_chunks % 2 == 0 and n_chunks >= 4
    mesh = plsc.VectorSubcoreMesh(core_axis_name="c", subcore_axis_name="s")

    @functools.partial(
        pl.kernel,
        mesh=mesh,
        out_type=jax.ShapeDtypeStruct((n_batch, n_time, _W), jnp.float32),
        scratch_types=[
            pltpu.VMEM((chunk,), jnp.int32),
            pltpu.VMEM((chunk,), jnp.int32),
            pltpu.VMEM((chunk,), jnp.int32),
            pltpu.VMEM((chunk,), jnp.int32),
            pltpu.VMEM((chunk, _D), jnp.float32),
            pltpu.VMEM((chunk, _D), jnp.float32),
            pltpu.SemaphoreType.DMA,
            pltpu.SemaphoreType.DMA,
            pltpu.SemaphoreType.DMA,
            pltpu.SemaphoreType.DMA,
            pltpu.SemaphoreType.DMA,
            pltpu.SemaphoreType.DMA,
        ],
        compiler_params=pltpu.CompilerParams(use_tc_tiling_on_sc=False),
    )
    def gather_kernel(idx_hbm, table_hbm, out_hbm,
                      i0, i1, q0, q1, r0, r1, is0, is1, gs0, gs1, ss0, ss1):
        ibuf, qbuf, rbuf = (i0, i1), (q0, q1), (r0, r1)
        isem, gsem, ssem = (is0, is1), (gs0, gs1), (ss0, ss1)
        wid = lax.axis_index("s") * _NC + lax.axis_index("c")
        base = wid * per_w

        def off(g):
            return base + g * chunk

        def idx_start(g, b):
            pltpu.async_copy(idx_hbm.at[pl.ds(off(g), chunk)], ibuf[b], isem[b])

        def idx_wait(b):
            pltpu.make_async_copy(
                idx_hbm.at[pl.ds(base, chunk)], ibuf[b], isem[b]).wait()

        def gather_start(b):
            def dbl(j, carry):
                sl = pl.ds(j * _LANES, _LANES)
                qbuf[b][sl] = ibuf[b][sl] * 2
                return carry

            lax.fori_loop(0, chunk // _LANES, dbl, 0)
            pltpu.async_copy(table_hbm.at[qbuf[b]], rbuf[b], gsem[b])

        def gather_wait(b):
            pltpu.make_async_copy(
                table_hbm.at[qbuf[b]], rbuf[b], gsem[b]).wait()

        def scatter_start(g, b):
            row0 = wid * rows_w + g * chunk_rows
            for k in range(chunk_rows):
                pltpu.async_copy(
                    rbuf[b].at[pl.ds(k * n_time, n_time)],
                    out_hbm.at[row0 + k, :, pl.ds(0, _D)], ssem[b])

        def scatter_wait(b):
            for _ in range(chunk_rows):
                pltpu.make_async_copy(
                    rbuf[b].at[pl.ds(0, n_time)],
                    out_hbm.at[0, :, pl.ds(0, _D)], ssem[b]).wait()

        idx_start(0, 0)
        idx_start(1, 1)
        idx_wait(0)
        gather_start(0)

        def body(i, carry):
            for b in range(2):
                g = 2 * i + b
                nb = 1 - b

                @pl.when(g + 1 < n_chunks)
                def _():
                    idx_wait(nb)

                    @pl.when(g >= 1)
                    def _():
                        scatter_wait(nb)

                    gather_start(nb)

                gather_wait(b)

                @pl.when(g + 2 < n_chunks)
                def _():
                    idx_start(g + 2, b)

                scatter_start(g, b)
            return carry

        lax.fori_loop(0, n_chunks // 2, body, 0)
        scatter_wait(0)
        scatter_wait(1)

    return gather_kernel


def kernel(inp_tokens, emb_table):
    n_batch, n_time = inp_tokens.shape
    idx = inp_tokens.reshape(-1).astype(jnp.int32)
    table_dup = _make_stage(emb_table.shape[0])(emb_table.T)
    table_dup = table_dup.reshape(2 * emb_table.shape[0], _D)
    out128 = _make_gather(n_batch, n_time, 2)(idx, table_dup)
    return out128[:, :, :_D]


# speedup vs baseline: 1.7775x; 1.1024x over previous
import functools

import jax
import jax.numpy as jnp
from jax import lax
from jax.experimental import pallas as pl
from jax.experimental.pallas import tpu as pltpu
from jax.experimental.pallas import tpu_sc as plsc

_D = 64
_W = 2 * _D
_SCALE = 64 ** 0.5
_LANES = 16

_info = plsc.get_sparse_core_info()
_NC, _NS = _info.num_cores, _info.num_subcores
_NW = _NC * _NS


def _make_gather(n_batch: int, n_time: int, chunk_rows: int):
    n_idx = n_batch * n_time
    chunk = chunk_rows * n_time
    assert n_idx % (_NW * chunk) == 0
    per_w = n_idx // _NW
    rows_w = per_w // n_time
    n_chunks = per_w // chunk
    assert n_chunks % 2 == 0 and n_chunks >= 4
    mesh = plsc.VectorSubcoreMesh(core_axis_name="c", subcore_axis_name="s")

    @functools.partial(
        pl.kernel,
        mesh=mesh,
        out_type=jax.ShapeDtypeStruct((n_batch, n_time, _W), jnp.float32),
        scratch_types=[
            pltpu.VMEM((chunk,), jnp.int32),
            pltpu.VMEM((chunk,), jnp.int32),
            pltpu.VMEM((chunk,), jnp.int32),
            pltpu.VMEM((chunk,), jnp.int32),
            pltpu.VMEM((chunk, _D), jnp.float32),
            pltpu.VMEM((chunk, _D), jnp.float32),
            pltpu.SemaphoreType.DMA,
            pltpu.SemaphoreType.DMA,
            pltpu.SemaphoreType.DMA,
            pltpu.SemaphoreType.DMA,
            pltpu.SemaphoreType.DMA,
            pltpu.SemaphoreType.DMA,
        ],
        compiler_params=pltpu.CompilerParams(use_tc_tiling_on_sc=False),
    )
    def gather_kernel(idx_hbm, table_hbm, out_hbm,
                      i0, i1, q0, q1, r0, r1, is0, is1, gs0, gs1, ss0, ss1):
        ibuf, qbuf, rbuf = (i0, i1), (q0, q1), (r0, r1)
        isem, gsem, ssem = (is0, is1), (gs0, gs1), (ss0, ss1)
        wid = lax.axis_index("s") * _NC + lax.axis_index("c")
        base = wid * per_w

        def off(g):
            return base + g * chunk

        def idx_start(g, b):
            pltpu.async_copy(idx_hbm.at[pl.ds(off(g), chunk)], ibuf[b], isem[b])

        def idx_wait(b):
            pltpu.make_async_copy(
                idx_hbm.at[pl.ds(base, chunk)], ibuf[b], isem[b]).wait()

        def gather_start(b):
            def dbl(j, carry):
                sl = pl.ds(j * _LANES, _LANES)
                qbuf[b][sl] = ibuf[b][sl] * 2
                return carry

            lax.fori_loop(0, chunk // _LANES, dbl, 0)
            pltpu.async_copy(table_hbm.at[qbuf[b]], rbuf[b], gsem[b])

        def gather_wait(b):
            pltpu.make_async_copy(
                table_hbm.at[qbuf[b]], rbuf[b], gsem[b]).wait()

        def scatter_start(g, b):
            row0 = wid * rows_w + g * chunk_rows
            for k in range(chunk_rows):
                pltpu.async_copy(
                    rbuf[b].at[pl.ds(k * n_time, n_time)],
                    out_hbm.at[row0 + k, :, pl.ds(0, _D)], ssem[b])

        def scatter_wait(b):
            for _ in range(chunk_rows):
                pltpu.make_async_copy(
                    rbuf[b].at[pl.ds(0, n_time)],
                    out_hbm.at[0, :, pl.ds(0, _D)], ssem[b]).wait()

        def scale(b):
            rows = rbuf[b]

            @plsc.parallel_loop(0, chunk, 1, unroll=4)
            def _(r):
                for c in range(_D // _LANES):
                    sl = pl.ds(c * _LANES, _LANES)
                    rows[r, sl] = rows[r, sl] * _SCALE

        idx_start(0, 0)
        idx_start(1, 1)
        idx_wait(0)
        gather_start(0)

        def body(i, carry):
            for b in range(2):
                g = 2 * i + b
                nb = 1 - b

                @pl.when(g + 1 < n_chunks)
                def _():
                    idx_wait(nb)

                    @pl.when(g >= 1)
                    def _():
                        scatter_wait(nb)

                    gather_start(nb)

                gather_wait(b)

                @pl.when(g + 2 < n_chunks)
                def _():
                    idx_start(g + 2, b)

                scale(b)
                scatter_start(g, b)
            return carry

        lax.fori_loop(0, n_chunks // 2, body, 0)
        scatter_wait(0)
        scatter_wait(1)

    return gather_kernel


def kernel(inp_tokens, emb_table):
    n_batch, n_time = inp_tokens.shape
    idx = inp_tokens.reshape(-1).astype(jnp.int32)
    table_dup = jnp.pad(emb_table, ((0, 0), (0, _W - _D)))
    table_dup = table_dup.reshape(2 * emb_table.shape[0], _D)
    out128 = _make_gather(n_batch, n_time, 4)(idx, table_dup)
    return out128[:, :, :_D]


# speedup vs baseline: 1.7810x; 1.0020x over previous
import functools

import jax
import jax.numpy as jnp
from jax import lax
from jax.experimental import pallas as pl
from jax.experimental.pallas import tpu as pltpu
from jax.experimental.pallas import tpu_sc as plsc

_D = 64
_W = 2 * _D
_SCALE = 64 ** 0.5
_LANES = 16

_info = plsc.get_sparse_core_info()
_NC, _NS = _info.num_cores, _info.num_subcores
_NW = _NC * _NS


def _make_gather(n_batch: int, n_time: int, chunk_rows: int):
    n_idx = n_batch * n_time
    chunk = chunk_rows * n_time
    assert n_idx % (_NW * chunk) == 0
    per_w = n_idx // _NW
    rows_w = per_w // n_time
    n_chunks = per_w // chunk
    assert n_chunks % 2 == 0 and n_chunks >= 4
    mesh = plsc.VectorSubcoreMesh(core_axis_name="c", subcore_axis_name="s")

    @functools.partial(
        pl.kernel,
        mesh=mesh,
        out_type=jax.ShapeDtypeStruct((n_batch, n_time, _W), jnp.float32),
        scratch_types=[
            pltpu.VMEM((chunk,), jnp.int32),
            pltpu.VMEM((chunk,), jnp.int32),
            pltpu.VMEM((chunk,), jnp.int32),
            pltpu.VMEM((chunk,), jnp.int32),
            pltpu.VMEM((chunk, _D), jnp.float32),
            pltpu.VMEM((chunk, _D), jnp.float32),
            pltpu.SemaphoreType.DMA,
            pltpu.SemaphoreType.DMA,
            pltpu.SemaphoreType.DMA,
            pltpu.SemaphoreType.DMA,
            pltpu.SemaphoreType.DMA,
            pltpu.SemaphoreType.DMA,
        ],
        compiler_params=pltpu.CompilerParams(use_tc_tiling_on_sc=False),
    )
    def gather_kernel(idx_hbm, table_hbm, out_hbm,
                      i0, i1, q0, q1, r0, r1, is0, is1, gs0, gs1, ss0, ss1):
        ibuf, qbuf, rbuf = (i0, i1), (q0, q1), (r0, r1)
        isem, gsem, ssem = (is0, is1), (gs0, gs1), (ss0, ss1)
        wid = lax.axis_index("s") * _NC + lax.axis_index("c")
        base = wid * per_w

        def off(g):
            return base + g * chunk

        def idx_start(g, b):
            pltpu.async_copy(idx_hbm.at[pl.ds(off(g), chunk)], ibuf[b], isem[b])

        def idx_wait(b):
            pltpu.make_async_copy(
                idx_hbm.at[pl.ds(base, chunk)], ibuf[b], isem[b]).wait()

        def gather_start(b):
            def dbl(j, carry):
                sl = pl.ds(j * _LANES, _LANES)
                qbuf[b][sl] = ibuf[b][sl] * 2
                return carry

            lax.fori_loop(0, chunk // _LANES, dbl, 0)
            pltpu.async_copy(table_hbm.at[qbuf[b]], rbuf[b], gsem[b])

        def gather_wait(b):
            pltpu.make_async_copy(
                table_hbm.at[qbuf[b]], rbuf[b], gsem[b]).wait()

        def scatter_start(g, b):
            row0 = wid * rows_w + g * chunk_rows
            for k in range(chunk_rows):
                pltpu.async_copy(
                    rbuf[b].at[pl.ds(k * n_time, n_time)],
                    out_hbm.at[row0 + k, :, pl.ds(0, _D)], ssem[b])

        def scatter_wait(b):
            for _ in range(chunk_rows):
                pltpu.make_async_copy(
                    rbuf[b].at[pl.ds(0, n_time)],
                    out_hbm.at[0, :, pl.ds(0, _D)], ssem[b]).wait()

        def scale(b):
            rows = rbuf[b]

            @plsc.parallel_loop(0, chunk, 1, unroll=4)
            def _(r):
                for c in range(_D // _LANES):
                    sl = pl.ds(c * _LANES, _LANES)
                    rows[r, sl] = rows[r, sl] * _SCALE

        idx_start(0, 0)
        idx_start(1, 1)
        idx_wait(0)
        gather_start(0)

        def body(i, carry):
            for b in range(2):
                g = 2 * i + b
                nb = 1 - b

                @pl.when(g + 1 < n_chunks)
                def _():
                    idx_wait(nb)

                    @pl.when(g >= 1)
                    def _():
                        scatter_wait(nb)

                    gather_start(nb)

                gather_wait(b)

                @pl.when(g + 2 < n_chunks)
                def _():
                    idx_start(g + 2, b)

                scale(b)
                scatter_start(g, b)
            return carry

        lax.fori_loop(0, n_chunks // 2, body, 0)
        scatter_wait(0)
        scatter_wait(1)

    return gather_kernel


def kernel(inp_tokens, emb_table):
    n_batch, n_time = inp_tokens.shape
    idx = inp_tokens.reshape(-1).astype(jnp.int32)
    table_dup = jnp.pad(emb_table, ((0, 0), (0, _W - _D)))
    table_dup = table_dup.reshape(2 * emb_table.shape[0], _D)
    out128 = _make_gather(n_batch, n_time, 2)(idx, table_dup)
    return out128[:, :, :_D]
